# baseline (speedup 1.0000x reference)
import functools

import jax
import jax.numpy as jnp
from jax.experimental import pallas as pl
from jax.experimental.pallas import tpu as pltpu

_RADIUS = 0.035
_BQ = 128
_BS = 128
_H = 32


def _prep_body(xs_ref, w1_ref, b1_ref, a_ref, b_ref):
    xs_i = xs_ref[...]
    w1 = w1_ref[...]
    a_ref[...] = jnp.dot(xs_i, w1[0:2, :], preferred_element_type=jnp.float32)
    b_ref[...] = (jnp.dot(xs_i, w1[2:4, :], preferred_element_type=jnp.float32)
                  + b1_ref[...])


def _main_body(xs_ref, a_ref, bp_ref, xp_ref, yp_ref, fp_ref,
               tmin_ref, tmax_ref, w2blk_ref, b2row_ref, w3blk_ref, b3_ref,
               out_ref):
    xs_i = xs_ref[...]
    a_i = a_ref[...]
    xmin = xs_ref[0, 0]
    xmax = xs_ref[_BQ - 1, 0]

    lo = jnp.sum((tmax_ref[...] < xmin - _RADIUS).astype(jnp.int32))
    hi = jnp.sum((tmin_ref[...] <= xmax + _RADIUS).astype(jnp.int32))

    xi3 = xs_i[:, 0:1].reshape(_BQ, 1, 1)
    yi3 = xs_i[:, 1:2].reshape(_BQ, 1, 1)

    ap4 = jnp.concatenate([a_i, a_i, a_i, a_i], axis=1)

    w2blk = w2blk_ref[...]
    b2row = b2row_ref[...]
    w3blk = w3blk_ref[...]
    b3 = b3_ref[0, 0]

    def body(jt, carry):
        acc, deg = carry
        jr = jt * _H
        xj3 = xp_ref[pl.ds(jr, _H), :].reshape(1, _H, 4)
        yj3 = yp_ref[pl.ds(jr, _H), :].reshape(1, _H, 4)
        fj3 = fp_ref[pl.ds(jr, _H), :].reshape(1, _H, 4)
        xd = xi3 - xj3
        yd = yi3 - yj3
        mask3 = (xd * xd + yd * yd) <= _RADIUS * _RADIUS

        bp_j = bp_ref[pl.ds(jr, _H), :]
        h1p = (ap4[:, None, :] + bp_j[None, :, :]).reshape(_BQ * _H, 128)
        h1 = jax.nn.gelu(h1p)
        h2 = jax.nn.gelu(
            jnp.dot(h1, w2blk, preferred_element_type=jnp.float32) + b2row)
        k4 = jnp.dot(h2, w3blk, preferred_element_type=jnp.float32) + b3
        k3 = k4.reshape(_BQ, _H, 4)

        contrib = jnp.where(mask3, k3 * fj3, 0.0)
        m = jnp.where(mask3, 1.0, 0.0)
        return acc + contrib, deg + m

    acc0 = jnp.zeros((_BQ, _H, 4), jnp.float32)
    acc, deg = jax.lax.fori_loop(lo, hi, body, (acc0, acc0))
    accs = jnp.sum(jnp.sum(acc, axis=1), axis=1, keepdims=True)
    degs = jnp.sum(jnp.sum(deg, axis=1), axis=1, keepdims=True)
    out_ref[...] = accs / jnp.maximum(degs, 1.0)


def kernel(coords, samples, sigma, W1, b1, W2, b2, W3, b3):
    del sigma
    n = coords.shape[1]
    nb = -(-n // _BQ)
    n_pad = nb * _BQ
    ntp = -(-nb // 128) * 128

    xall = coords[0, :]
    perm = jnp.argsort(xall)
    xs = coords.T[perm]
    fs = samples[perm]
    pad = n_pad - n
    xs = jnp.concatenate(
        [xs, jnp.full((pad, 2), 2.0, jnp.float32)], axis=0)
    fs = jnp.concatenate([fs, jnp.zeros((pad,), jnp.float32)])

    xsort = xs[:, 0]
    ysort = xs[:, 1]
    xp = xsort.reshape(n_pad // 4, 4)
    yp = ysort.reshape(n_pad // 4, 4)
    fp = fs.reshape(n_pad // 4, 4)
    tmin = jnp.full((ntp,), 3.0, jnp.float32).at[:nb].set(xsort[::_BQ])
    tmax = jnp.full((ntp,), 3.0, jnp.float32).at[:nb].set(
        xsort[_BQ - 1::_BQ])

    a_all, b_all = pl.pallas_call(
        _prep_body,
        grid=(nb,),
        in_specs=[
            pl.BlockSpec((_BQ, 2), lambda i: (i, 0)),
            pl.BlockSpec((4, _H), lambda i: (0, 0)),
            pl.BlockSpec((1, _H), lambda i: (0, 0)),
        ],
        out_specs=[
            pl.BlockSpec((_BQ, _H), lambda i: (i, 0)),
            pl.BlockSpec((_BQ, _H), lambda i: (i, 0)),
        ],
        out_shape=[
            jax.ShapeDtypeStruct((n_pad, _H), jnp.float32),
            jax.ShapeDtypeStruct((n_pad, _H), jnp.float32),
        ],
    )(xs, W1, b1[None, :])

    bp = b_all.reshape(n_pad // 4, 128)
    eye4 = jnp.eye(4, dtype=jnp.float32)
    w2blk = jnp.kron(eye4, W2)
    b2row = jnp.tile(b2, (4,))[None, :]
    w3blk = jnp.kron(eye4, W3)
    b3s = b3.reshape(1, 1)

    full = lambda i: (0, 0)
    out = pl.pallas_call(
        _main_body,
        grid=(nb,),
        in_specs=[
            pl.BlockSpec((_BQ, 2), lambda i: (i, 0)),
            pl.BlockSpec((_BQ, _H), lambda i: (i, 0)),
            pl.BlockSpec((n_pad // 4, 128), full),
            pl.BlockSpec((n_pad // 4, 4), full),
            pl.BlockSpec((n_pad // 4, 4), full),
            pl.BlockSpec((n_pad // 4, 4), full),
            pl.BlockSpec((1, ntp), full),
            pl.BlockSpec((1, ntp), full),
            pl.BlockSpec((128, 128), full),
            pl.BlockSpec((1, 128), full),
            pl.BlockSpec((128, 4), full),
            pl.BlockSpec((1, 1), full),
        ],
        out_specs=pl.BlockSpec((_BQ, 1), lambda i: (i, 0)),
        out_shape=jax.ShapeDtypeStruct((n_pad, 1), jnp.float32),
        compiler_params=pltpu.CompilerParams(
            dimension_semantics=("parallel",)),
    )(xs, a_all, bp, xp, yp, fp, tmin[None, :], tmax[None, :],
      w2blk, b2row, w3blk, b3s)

    out_sorted = out[:n, 0]
    return jnp.zeros((n,), jnp.float32).at[perm].set(out_sorted)


# speedup vs baseline: 1.1097x; 1.1097x over previous
import functools

import jax
import jax.numpy as jnp
from jax.experimental import pallas as pl
from jax.experimental.pallas import tpu as pltpu

_RADIUS = 0.035
_BQ = 128
_BS = 128
_H = 32


def _prep_body(xs_ref, w1_ref, b1_ref, a_ref, b_ref):
    xs_i = xs_ref[...]
    w1 = w1_ref[...]
    a_ref[...] = jnp.dot(xs_i, w1[0:2, :], preferred_element_type=jnp.float32)
    b_ref[...] = (jnp.dot(xs_i, w1[2:4, :], preferred_element_type=jnp.float32)
                  + b1_ref[...])


def _main_body(xs_ref, a_ref, bp_ref, xp_ref, yp_ref, fp_ref,
               tmin_ref, tmax_ref, w2blk_ref, b2row_ref, w3blk_ref, b3_ref,
               out_ref):
    xs_i = xs_ref[...]
    a_i = a_ref[...]
    xmin = xs_ref[0, 0]
    xmax = xs_ref[_BQ - 1, 0]

    lo = jnp.sum((tmax_ref[...] < xmin - _RADIUS).astype(jnp.int32))
    hi = jnp.sum((tmin_ref[...] <= xmax + _RADIUS).astype(jnp.int32))

    xi3 = xs_i[:, 0:1].reshape(_BQ, 1, 1)
    yi3 = xs_i[:, 1:2].reshape(_BQ, 1, 1)

    ap4 = jnp.concatenate([a_i, a_i, a_i, a_i], axis=1)

    w2blk = w2blk_ref[...]
    b2row = b2row_ref[...]
    w3blk = w3blk_ref[...]
    b3 = b3_ref[0, 0]

    def body(jt, carry):
        acc, deg = carry
        jr = jt * _H
        xj3 = xp_ref[pl.ds(jr, _H), :].reshape(1, _H, 4)
        yj3 = yp_ref[pl.ds(jr, _H), :].reshape(1, _H, 4)
        fj3 = fp_ref[pl.ds(jr, _H), :].reshape(1, _H, 4)
        xd = xi3 - xj3
        yd = yi3 - yj3
        mask3 = (xd * xd + yd * yd) <= _RADIUS * _RADIUS

        bp_j = bp_ref[pl.ds(jr, _H), :]
        h1p = (ap4[:, None, :] + bp_j[None, :, :]).reshape(_BQ * _H, 128)
        h1 = jax.nn.gelu(h1p)
        h2 = jax.nn.gelu(
            jnp.dot(h1, w2blk, preferred_element_type=jnp.float32) + b2row)
        k4 = jnp.dot(h2, w3blk, preferred_element_type=jnp.float32) + b3
        k3 = k4.reshape(_BQ, _H, 4)

        contrib = jnp.where(mask3, k3 * fj3, 0.0)
        m = jnp.where(mask3, 1.0, 0.0)
        acc = acc + jnp.sum(contrib, axis=1)
        deg = deg + jnp.sum(m, axis=1)
        return acc, deg

    acc0 = jnp.zeros((_BQ, 4), jnp.float32)
    acc, deg = jax.lax.fori_loop(lo, hi, body, (acc0, acc0))
    accs = jnp.sum(acc, axis=1, keepdims=True)
    degs = jnp.sum(deg, axis=1, keepdims=True)
    out_ref[...] = accs / jnp.maximum(degs, 1.0)


def kernel(coords, samples, sigma, W1, b1, W2, b2, W3, b3):
    del sigma
    n = coords.shape[1]
    nb = -(-n // _BQ)
    n_pad = nb * _BQ
    ntp = -(-nb // 128) * 128

    xall = coords[0, :]
    perm = jnp.argsort(xall)
    xs = coords.T[perm]
    fs = samples[perm]
    pad = n_pad - n
    xs = jnp.concatenate(
        [xs, jnp.full((pad, 2), 2.0, jnp.float32)], axis=0)
    fs = jnp.concatenate([fs, jnp.zeros((pad,), jnp.float32)])

    xsort = xs[:, 0]
    ysort = xs[:, 1]
    xp = xsort.reshape(n_pad // 4, 4)
    yp = ysort.reshape(n_pad // 4, 4)
    fp = fs.reshape(n_pad // 4, 4)
    tmin = jnp.full((ntp,), 3.0, jnp.float32).at[:nb].set(xsort[::_BQ])
    tmax = jnp.full((ntp,), 3.0, jnp.float32).at[:nb].set(
        xsort[_BQ - 1::_BQ])

    a_all, b_all = pl.pallas_call(
        _prep_body,
        grid=(nb,),
        in_specs=[
            pl.BlockSpec((_BQ, 2), lambda i: (i, 0)),
            pl.BlockSpec((4, _H), lambda i: (0, 0)),
            pl.BlockSpec((1, _H), lambda i: (0, 0)),
        ],
        out_specs=[
            pl.BlockSpec((_BQ, _H), lambda i: (i, 0)),
            pl.BlockSpec((_BQ, _H), lambda i: (i, 0)),
        ],
        out_shape=[
            jax.ShapeDtypeStruct((n_pad, _H), jnp.float32),
            jax.ShapeDtypeStruct((n_pad, _H), jnp.float32),
        ],
    )(xs, W1, b1[None, :])

    bp = b_all.reshape(n_pad // 4, 128)
    eye4 = jnp.eye(4, dtype=jnp.float32)
    w2blk = jnp.kron(eye4, W2)
    b2row = jnp.tile(b2, (4,))[None, :]
    w3blk = jnp.kron(eye4, W3)
    b3s = b3.reshape(1, 1)

    full = lambda i: (0, 0)
    out = pl.pallas_call(
        _main_body,
        grid=(nb,),
        in_specs=[
            pl.BlockSpec((_BQ, 2), lambda i: (i, 0)),
            pl.BlockSpec((_BQ, _H), lambda i: (i, 0)),
            pl.BlockSpec((n_pad // 4, 128), full),
            pl.BlockSpec((n_pad // 4, 4), full),
            pl.BlockSpec((n_pad // 4, 4), full),
            pl.BlockSpec((n_pad // 4, 4), full),
            pl.BlockSpec((1, ntp), full),
            pl.BlockSpec((1, ntp), full),
            pl.BlockSpec((128, 128), full),
            pl.BlockSpec((1, 128), full),
            pl.BlockSpec((128, 4), full),
            pl.BlockSpec((1, 1), full),
        ],
        out_specs=pl.BlockSpec((_BQ, 1), lambda i: (i, 0)),
        out_shape=jax.ShapeDtypeStruct((n_pad, 1), jnp.float32),
        compiler_params=pltpu.CompilerParams(
            dimension_semantics=("parallel",)),
    )(xs, a_all, bp, xp, yp, fp, tmin[None, :], tmax[None, :],
      w2blk, b2row, w3blk, b3s)

    out_sorted = out[:n, 0]
    return jnp.zeros((n,), jnp.float32).at[perm].set(out_sorted)


# speedup vs baseline: 1.1266x; 1.0153x over previous
import functools

import jax
import jax.numpy as jnp
from jax.experimental import pallas as pl
from jax.experimental.pallas import tpu as pltpu

_RADIUS = 0.035
_BQ = 128
_BS = 128
_H = 32


def _prep_body(xs_ref, w1_ref, b1_ref, a_ref, b_ref):
    xs_i = xs_ref[...]
    w1 = w1_ref[...]
    a_ref[...] = jnp.dot(xs_i, w1[0:2, :], preferred_element_type=jnp.float32)
    b_ref[...] = (jnp.dot(xs_i, w1[2:4, :], preferred_element_type=jnp.float32)
                  + b1_ref[...])


def _main_body(xs_ref, a_ref, bp_ref, xp_ref, yp_ref, fp_ref,
               tmin_ref, tmax_ref, w2blk_ref, b2row_ref, w3blk_ref, b3_ref,
               out_ref):
    xs_i = xs_ref[...]
    a_i = a_ref[...]
    xmin = xs_ref[0, 0]
    xmax = xs_ref[_BQ - 1, 0]

    lo = jnp.sum((tmax_ref[...] < xmin - _RADIUS).astype(jnp.int32))
    hi = jnp.sum((tmin_ref[...] <= xmax + _RADIUS).astype(jnp.int32))

    xi3 = xs_i[:, 0:1].reshape(_BQ, 1, 1)
    yi3 = xs_i[:, 1:2].reshape(_BQ, 1, 1)

    ap4 = jnp.concatenate([a_i, a_i, a_i, a_i], axis=1)
    a_part = jnp.broadcast_to(ap4[:, None, :], (_BQ, _H, 128))
    a_part = a_part.reshape(_BQ * _H, 128)

    w2blk = w2blk_ref[...]
    b2row = b2row_ref[...]
    w3blk = w3blk_ref[...]
    b3 = b3_ref[0, 0]

    def body(jt, carry):
        acc, deg = carry
        jr = jt * _H
        xj3 = xp_ref[pl.ds(jr, _H), :].reshape(1, _H, 4)
        yj3 = yp_ref[pl.ds(jr, _H), :].reshape(1, _H, 4)
        fj3 = fp_ref[pl.ds(jr, _H), :].reshape(1, _H, 4)
        xd = xi3 - xj3
        yd = yi3 - yj3
        mask3 = (xd * xd + yd * yd) <= _RADIUS * _RADIUS

        bp_j = bp_ref[pl.ds(jr, _H), :]
        b_part = jnp.broadcast_to(bp_j[None, :, :], (_BQ, _H, 128))
        b_part = b_part.reshape(_BQ * _H, 128)

        h1 = jax.nn.gelu(a_part + b_part)
        h2 = jax.nn.gelu(
            jnp.dot(h1, w2blk, preferred_element_type=jnp.float32) + b2row)
        k4 = jnp.dot(h2, w3blk, preferred_element_type=jnp.float32) + b3
        k3 = k4.reshape(_BQ, _H, 4)

        contrib = jnp.where(mask3, k3 * fj3, 0.0)
        m = jnp.where(mask3, 1.0, 0.0)
        acc = acc + jnp.sum(contrib, axis=1)
        deg = deg + jnp.sum(m, axis=1)
        return acc, deg

    acc0 = jnp.zeros((_BQ, 4), jnp.float32)
    acc, deg = jax.lax.fori_loop(lo, hi, body, (acc0, acc0))
    accs = jnp.sum(acc, axis=1, keepdims=True)
    degs = jnp.sum(deg, axis=1, keepdims=True)
    out_ref[...] = accs / jnp.maximum(degs, 1.0)


def kernel(coords, samples, sigma, W1, b1, W2, b2, W3, b3):
    del sigma
    n = coords.shape[1]
    nb = -(-n // _BQ)
    n_pad = nb * _BQ
    ntp = -(-nb // 128) * 128

    xall = coords[0, :]
    perm = jnp.argsort(xall)
    xs = coords.T[perm]
    fs = samples[perm]
    pad = n_pad - n
    xs = jnp.concatenate(
        [xs, jnp.full((pad, 2), 2.0, jnp.float32)], axis=0)
    fs = jnp.concatenate([fs, jnp.zeros((pad,), jnp.float32)])

    xsort = xs[:, 0]
    ysort = xs[:, 1]
    xp = xsort.reshape(n_pad // 4, 4)
    yp = ysort.reshape(n_pad // 4, 4)
    fp = fs.reshape(n_pad // 4, 4)
    tmin = jnp.full((ntp,), 3.0, jnp.float32).at[:nb].set(xsort[::_BQ])
    tmax = jnp.full((ntp,), 3.0, jnp.float32).at[:nb].set(
        xsort[_BQ - 1::_BQ])

    a_all, b_all = pl.pallas_call(
        _prep_body,
        grid=(nb,),
        in_specs=[
            pl.BlockSpec((_BQ, 2), lambda i: (i, 0)),
            pl.BlockSpec((4, _H), lambda i: (0, 0)),
            pl.BlockSpec((1, _H), lambda i: (0, 0)),
        ],
        out_specs=[
            pl.BlockSpec((_BQ, _H), lambda i: (i, 0)),
            pl.BlockSpec((_BQ, _H), lambda i: (i, 0)),
        ],
        out_shape=[
            jax.ShapeDtypeStruct((n_pad, _H), jnp.float32),
            jax.ShapeDtypeStruct((n_pad, _H), jnp.float32),
        ],
    )(xs, W1, b1[None, :])

    bp = b_all.reshape(n_pad // 4, 128)
    eye4 = jnp.eye(4, dtype=jnp.float32)
    w2blk = jnp.kron(eye4, W2)
    b2row = jnp.tile(b2, (4,))[None, :]
    w3blk = jnp.kron(eye4, W3)
    b3s = b3.reshape(1, 1)

    full = lambda i: (0, 0)
    out = pl.pallas_call(
        _main_body,
        grid=(nb,),
        in_specs=[
            pl.BlockSpec((_BQ, 2), lambda i: (i, 0)),
            pl.BlockSpec((_BQ, _H), lambda i: (i, 0)),
            pl.BlockSpec((n_pad // 4, 128), full),
            pl.BlockSpec((n_pad // 4, 4), full),
            pl.BlockSpec((n_pad // 4, 4), full),
            pl.BlockSpec((n_pad // 4, 4), full),
            pl.BlockSpec((1, ntp), full),
            pl.BlockSpec((1, ntp), full),
            pl.BlockSpec((128, 128), full),
            pl.BlockSpec((1, 128), full),
            pl.BlockSpec((128, 4), full),
            pl.BlockSpec((1, 1), full),
        ],
        out_specs=pl.BlockSpec((_BQ, 1), lambda i: (i, 0)),
        out_shape=jax.ShapeDtypeStruct((n_pad, 1), jnp.float32),
        compiler_params=pltpu.CompilerParams(
            dimension_semantics=("parallel",)),
    )(xs, a_all, bp, xp, yp, fp, tmin[None, :], tmax[None, :],
      w2blk, b2row, w3blk, b3s)

    out_sorted = out[:n, 0]
    return jnp.zeros((n,), jnp.float32).at[perm].set(out_sorted)
